# baseline (speedup 1.0000x reference)
import jax
import jax.numpy as jnp
from jax import lax
from jax.experimental import pallas as pl
from jax.experimental.pallas import tpu as pltpu
from jax.experimental.pallas import tpu_sc as plsc

N, E, D = 10000, 320000, 128
DW = D // 2
NC, NS, L = 2, 16, 16
NW = NC * NS
E_PER_W = E // NW
CHUNK = 400
N_CHUNKS = E_PER_W // CHUNK
N_PAIRS = N_CHUNKS // 2

_bc = lax.bitcast_convert_type
_MHI = -65536


def _mul_chunk(rs, rd, prod):
    def row_body(r, carry):
        for j in range(DW // L):
            sl = pl.ds(j * L, L)
            ws = rs[r, sl]
            wd = rd[r, sl]
            ae = _bc(ws << 16, jnp.float32)
            ao = _bc(ws & _MHI, jnp.float32)
            be = _bc(wd << 16, jnp.float32)
            bo = _bc(wd & _MHI, jnp.float32)
            re = _bc(ae * be, jnp.int32) + 0x8000
            ro = _bc(ao * bo, jnp.int32) + 0x8000
            prod[r, sl] = lax.shift_right_logical(re, 16) | (ro & _MHI)
        return carry

    lax.fori_loop(0, CHUNK, row_body, None, unroll=False)


def _sc_gather_mul(src_hbm, dst_hbm, z_hbm, x_hbm, ix_s, ix_d,
                   rsA, rdA, rsB, rdB,
                   gsem_a, gsem_b, osem_a, osem_b):
    sid = lax.axis_index("s")
    wid = sid * NC + lax.axis_index("c")
    base = wid * E_PER_W

    pltpu.sync_copy(src_hbm.at[pl.ds(base, E_PER_W)], ix_s)
    pltpu.sync_copy(dst_hbm.at[pl.ds(base, E_PER_W)], ix_d)

    def gather(c, rs, rd, sem):
        off = c * CHUNK
        pltpu.async_copy(z_hbm.at[ix_s.at[pl.ds(off, CHUNK)]], rs, sem)
        pltpu.async_copy(z_hbm.at[ix_d.at[pl.ds(off, CHUNK)]], rd, sem)

    def drain_gather(rs, rd, sem):
        pltpu.make_async_copy(z_hbm.at[ix_s.at[pl.ds(0, CHUNK)]], rs,
                              sem).wait()
        pltpu.make_async_copy(z_hbm.at[ix_d.at[pl.ds(0, CHUNK)]], rd,
                              sem).wait()

    def drain_out(rs, sem):
        pltpu.make_async_copy(rs, x_hbm.at[pl.ds(0, CHUNK)], sem).wait()

    gather(0, rsA, rdA, gsem_a)

    def pair_body(k, carry):
        c0 = 2 * k

        @pl.when(k > 0)
        def _():
            drain_out(rsB, osem_b)

        gather(c0 + 1, rsB, rdB, gsem_b)

        drain_gather(rsA, rdA, gsem_a)
        _mul_chunk(rsA, rdA, rsA)
        pltpu.async_copy(rsA, x_hbm.at[pl.ds(base + c0 * CHUNK, CHUNK)],
                         osem_a)

        drain_gather(rsB, rdB, gsem_b)
        _mul_chunk(rsB, rdB, rsB)
        pltpu.async_copy(rsB, x_hbm.at[pl.ds(base + (c0 + 1) * CHUNK, CHUNK)],
                         osem_b)

        drain_out(rsA, osem_a)
        gather(c0 + 2, rsA, rdA, gsem_a)
        return carry

    lax.fori_loop(0, N_PAIRS, pair_body, None, unroll=False)

    ct = N_CHUNKS - 1
    drain_gather(rsA, rdA, gsem_a)
    _mul_chunk(rsA, rdA, rsA)
    pltpu.async_copy(rsA, x_hbm.at[pl.ds(base + ct * CHUNK, CHUNK)], osem_a)
    drain_out(rsA, osem_a)
    drain_out(rsB, osem_b)


def _gather_mul(zw, src, dst):
    mesh = plsc.VectorSubcoreMesh(core_axis_name="c", subcore_axis_name="s",
                                  num_cores=NC, num_subcores=NS)
    return pl.kernel(
        _sc_gather_mul,
        out_type=jax.ShapeDtypeStruct((E, DW), jnp.int32),
        mesh=mesh,
        compiler_params=pltpu.CompilerParams(use_tc_tiling_on_sc=False),
        scratch_types=[
            pltpu.VMEM((E_PER_W,), jnp.int32),
            pltpu.VMEM((E_PER_W,), jnp.int32),
            pltpu.VMEM((CHUNK, DW), jnp.int32),
            pltpu.VMEM((CHUNK, DW), jnp.int32),
            pltpu.VMEM((CHUNK, DW), jnp.int32),
            pltpu.VMEM((CHUNK, DW), jnp.int32),
            pltpu.SemaphoreType.DMA,
            pltpu.SemaphoreType.DMA,
            pltpu.SemaphoreType.DMA,
            pltpu.SemaphoreType.DMA,
        ],
    )(src, dst, zw)


E_BLK = 3200


def _tc_mlp(xw_ref, w1a_ref, w1b_ref, b1_ref, w2_ref, b2_ref, o_ref):
    w = xw_ref[...]
    xe = _bc(w << 16, jnp.float32).astype(jnp.bfloat16)
    xo = _bc(w & _MHI, jnp.float32).astype(jnp.bfloat16)
    h = jnp.dot(xe, w1a_ref[...], preferred_element_type=jnp.float32)
    h += jnp.dot(xo, w1b_ref[...], preferred_element_type=jnp.float32)
    h = jnp.maximum(h + b1_ref[...], 0.0)
    o = jnp.sum(h * w2_ref[...], axis=1, keepdims=True) + b2_ref[...]
    o_ref[...] = o


def _mlp(xw, W1a, W1b, b1, W2, b2):
    grid = (E // E_BLK,)
    return pl.pallas_call(
        _tc_mlp,
        grid=grid,
        in_specs=[
            pl.BlockSpec((E_BLK, DW), lambda i: (i, 0)),
            pl.BlockSpec((DW, D), lambda i: (0, 0)),
            pl.BlockSpec((DW, D), lambda i: (0, 0)),
            pl.BlockSpec((1, D), lambda i: (0, 0)),
            pl.BlockSpec((1, D), lambda i: (0, 0)),
            pl.BlockSpec((1, 1), lambda i: (0, 0)),
        ],
        out_specs=pl.BlockSpec((E_BLK, 1), lambda i: (i, 0)),
        out_shape=jax.ShapeDtypeStruct((E, 1), jnp.float32),
    )(xw, W1a, W1b, b1.reshape(1, D), W2.reshape(1, D), b2.reshape(1, 1))


def kernel(z, edge, W1, b1, W2, b2):
    zb = z.astype(jnp.bfloat16)
    lo = _bc(zb[:, :DW], jnp.uint16).astype(jnp.int32)
    hi = _bc(zb[:, DW:], jnp.uint16).astype(jnp.int32)
    zw = lo | (hi << 16)
    xw = _gather_mul(zw, edge[0], edge[1])
    W1a = W1[:DW, :].astype(jnp.bfloat16)
    W1b = W1[DW:, :].astype(jnp.bfloat16)
    return _mlp(xw, W1a, W1b, b1, W2, b2)


# speedup vs baseline: 1.1194x; 1.1194x over previous
import jax
import jax.numpy as jnp
from jax import lax
from jax.experimental import pallas as pl
from jax.experimental.pallas import tpu as pltpu
from jax.experimental.pallas import tpu_sc as plsc

N, E, D = 10000, 320000, 128
DW = D // 2
NC, NS, L = 2, 16, 16
NW = NC * NS
E_PER_W = E // NW
CHUNK = 200
N_CHUNKS = E_PER_W // CHUNK
N_PAIRS = N_CHUNKS // 2

_bc = lax.bitcast_convert_type
_MHI = -65536


def _mul_chunk(rs, rd, prod):
    def row_body(r, carry):
        for j in range(DW // L):
            sl = pl.ds(j * L, L)
            ws = rs[r, sl]
            wd = rd[r, sl]
            ae = _bc(ws << 16, jnp.float32)
            ao = _bc(ws & _MHI, jnp.float32)
            be = _bc(wd << 16, jnp.float32)
            bo = _bc(wd & _MHI, jnp.float32)
            re = _bc(ae * be, jnp.int32) + 0x8000
            ro = _bc(ao * bo, jnp.int32) + 0x8000
            prod[r, sl] = lax.shift_right_logical(re, 16) | (ro & _MHI)
        return carry

    lax.fori_loop(0, CHUNK, row_body, None, unroll=False)


def _sc_gather_mul(src_hbm, dst_hbm, z_hbm, x_hbm,
                   isA, idA, isB, idB,
                   rsA, rdA, prA, rsB, rdB, prB, z_sh,
                   isem_a, isem_b, gsem_a, gsem_b, osem_a, osem_b):
    sid = lax.axis_index("s")
    wid = sid * NC + lax.axis_index("c")
    base = wid * E_PER_W

    rows_per_sub = N // NS
    r0 = sid * rows_per_sub
    pltpu.sync_copy(z_hbm.at[pl.ds(r0, rows_per_sub)],
                    z_sh.at[pl.ds(r0, rows_per_sub)])
    plsc.subcore_barrier()

    def issue_idx(c, i_s, i_d, sem):
        off = base + c * CHUNK
        pltpu.async_copy(src_hbm.at[pl.ds(off, CHUNK)], i_s, sem)
        pltpu.async_copy(dst_hbm.at[pl.ds(off, CHUNK)], i_d, sem)

    def drain_idx(i_s, i_d, sem):
        pltpu.make_async_copy(src_hbm.at[pl.ds(0, CHUNK)], i_s, sem).wait()
        pltpu.make_async_copy(dst_hbm.at[pl.ds(0, CHUNK)], i_d, sem).wait()

    def gather(i_s, i_d, rs, rd, sem):
        pltpu.async_copy(z_sh.at[i_s], rs, sem)
        pltpu.async_copy(z_sh.at[i_d], rd, sem)

    def drain_gather(i_s, i_d, rs, rd, sem):
        pltpu.make_async_copy(z_sh.at[i_s], rs, sem).wait()
        pltpu.make_async_copy(z_sh.at[i_d], rd, sem).wait()

    def drain_out(pr, sem):
        pltpu.make_async_copy(pr, x_hbm.at[pl.ds(0, CHUNK)], sem).wait()

    issue_idx(0, isA, idA, isem_a)
    issue_idx(1, isB, idB, isem_b)
    drain_idx(isA, idA, isem_a)
    gather(isA, idA, rsA, rdA, gsem_a)

    def pair_body(k, carry):
        c0 = 2 * k
        last = N_PAIRS - 1

        drain_idx(isB, idB, isem_b)
        gather(isB, idB, rsB, rdB, gsem_b)

        drain_gather(isA, idA, rsA, rdA, gsem_a)

        @pl.when(k < last)
        def _():
            issue_idx(c0 + 2, isA, idA, isem_a)

        @pl.when(k > 0)
        def _():
            drain_out(prA, osem_a)

        _mul_chunk(rsA, rdA, prA)
        pltpu.async_copy(prA, x_hbm.at[pl.ds(base + c0 * CHUNK, CHUNK)],
                         osem_a)

        @pl.when(k < last)
        def _():
            drain_idx(isA, idA, isem_a)
            gather(isA, idA, rsA, rdA, gsem_a)

        drain_gather(isB, idB, rsB, rdB, gsem_b)

        @pl.when(k < last)
        def _():
            issue_idx(c0 + 3, isB, idB, isem_b)

        @pl.when(k > 0)
        def _():
            drain_out(prB, osem_b)

        _mul_chunk(rsB, rdB, prB)
        pltpu.async_copy(prB, x_hbm.at[pl.ds(base + (c0 + 1) * CHUNK, CHUNK)],
                         osem_b)
        return carry

    lax.fori_loop(0, N_PAIRS, pair_body, None, unroll=False)
    drain_out(prA, osem_a)
    drain_out(prB, osem_b)


def _gather_mul(zw, src, dst):
    mesh = plsc.VectorSubcoreMesh(core_axis_name="c", subcore_axis_name="s",
                                  num_cores=NC, num_subcores=NS)
    return pl.kernel(
        _sc_gather_mul,
        out_type=jax.ShapeDtypeStruct((E, DW), jnp.int32),
        mesh=mesh,
        compiler_params=pltpu.CompilerParams(use_tc_tiling_on_sc=False),
        scratch_types=[
            pltpu.VMEM((CHUNK,), jnp.int32),
            pltpu.VMEM((CHUNK,), jnp.int32),
            pltpu.VMEM((CHUNK,), jnp.int32),
            pltpu.VMEM((CHUNK,), jnp.int32),
            pltpu.VMEM((CHUNK, DW), jnp.int32),
            pltpu.VMEM((CHUNK, DW), jnp.int32),
            pltpu.VMEM((CHUNK, DW), jnp.int32),
            pltpu.VMEM((CHUNK, DW), jnp.int32),
            pltpu.VMEM((CHUNK, DW), jnp.int32),
            pltpu.VMEM((CHUNK, DW), jnp.int32),
            pltpu.VMEM_SHARED((N, DW), jnp.int32),
            pltpu.SemaphoreType.DMA,
            pltpu.SemaphoreType.DMA,
            pltpu.SemaphoreType.DMA,
            pltpu.SemaphoreType.DMA,
            pltpu.SemaphoreType.DMA,
            pltpu.SemaphoreType.DMA,
        ],
    )(src, dst, zw)


E_BLK = 3200


def _tc_mlp(xw_ref, w1a_ref, w1b_ref, b1_ref, w2_ref, b2_ref, o_ref):
    w = xw_ref[...]
    xe = _bc(w << 16, jnp.float32).astype(jnp.bfloat16)
    xo = _bc(w & _MHI, jnp.float32).astype(jnp.bfloat16)
    h = jnp.dot(xe, w1a_ref[...], preferred_element_type=jnp.float32)
    h += jnp.dot(xo, w1b_ref[...], preferred_element_type=jnp.float32)
    h = jnp.maximum(h + b1_ref[...], 0.0)
    o = jnp.sum(h * w2_ref[...], axis=1, keepdims=True) + b2_ref[...]
    o_ref[...] = o


def _mlp(xw, W1a, W1b, b1, W2, b2):
    grid = (E // E_BLK,)
    return pl.pallas_call(
        _tc_mlp,
        grid=grid,
        in_specs=[
            pl.BlockSpec((E_BLK, DW), lambda i: (i, 0)),
            pl.BlockSpec((DW, D), lambda i: (0, 0)),
            pl.BlockSpec((DW, D), lambda i: (0, 0)),
            pl.BlockSpec((1, D), lambda i: (0, 0)),
            pl.BlockSpec((1, D), lambda i: (0, 0)),
            pl.BlockSpec((1, 1), lambda i: (0, 0)),
        ],
        out_specs=pl.BlockSpec((E_BLK, 1), lambda i: (i, 0)),
        out_shape=jax.ShapeDtypeStruct((E, 1), jnp.float32),
    )(xw, W1a, W1b, b1.reshape(1, D), W2.reshape(1, D), b2.reshape(1, 1))


def kernel(z, edge, W1, b1, W2, b2):
    zb = z.astype(jnp.bfloat16)
    lo = _bc(zb[:, :DW], jnp.uint16).astype(jnp.int32)
    hi = _bc(zb[:, DW:], jnp.uint16).astype(jnp.int32)
    zw = lo | (hi << 16)
    xw = _gather_mul(zw, edge[0], edge[1])
    W1a = W1[:DW, :].astype(jnp.bfloat16)
    W1b = W1[DW:, :].astype(jnp.bfloat16)
    return _mlp(xw, W1a, W1b, b1, W2, b2)


# speedup vs baseline: 1.4161x; 1.2651x over previous
import jax
import jax.numpy as jnp
from jax import lax
from jax.experimental import pallas as pl
from jax.experimental.pallas import tpu as pltpu
from jax.experimental.pallas import tpu_sc as plsc

N, E, D = 10000, 320000, 128
DW = D // 2
NC, NS, L = 2, 16, 16
NW = NC * NS
E_PER_W = E // NW
CHUNK = 200
N_CHUNKS = E_PER_W // CHUNK
N_PAIRS = N_CHUNKS // 2

_bc = lax.bitcast_convert_type
_MHI = -65536


def _mul_chunk(rs, rd, prod):
    def row_body(rr, carry):
        for h in range(2):
            r = 2 * rr + h
            for j in range(DW // L):
                sl = pl.ds(j * L, L)
                ws = rs[r, sl]
                wd = rd[r, sl]
                ae = _bc(ws << 16, jnp.float32)
                ao = _bc(ws & _MHI, jnp.float32)
                be = _bc(wd << 16, jnp.float32)
                bo = _bc(wd & _MHI, jnp.float32)
                re = _bc(ae * be, jnp.int32) + 0x8000
                ro = _bc(ao * bo, jnp.int32) + 0x8000
                prod[rr, pl.ds(h * DW + j * L, L)] = (
                    lax.shift_right_logical(re, 16) | (ro & _MHI))
        return carry

    lax.fori_loop(0, CHUNK // 2, row_body, None, unroll=False)


def _sc_gather_mul(src_hbm, dst_hbm, z_hbm, x_hbm,
                   isA, idA, isB, idB,
                   rsA, rdA, prA, rsB, rdB, prB, z_sh,
                   isem_a, isem_b, gsem_a, gsem_b, osem_a, osem_b):
    sid = lax.axis_index("s")
    wid = sid * NC + lax.axis_index("c")
    base = wid * E_PER_W

    rows_per_sub = N // NS
    r0 = sid * rows_per_sub
    pltpu.sync_copy(z_hbm.at[pl.ds(r0, rows_per_sub)],
                    z_sh.at[pl.ds(r0, rows_per_sub)])
    plsc.subcore_barrier()

    def issue_idx(c, i_s, i_d, sem):
        off = base + c * CHUNK
        pltpu.async_copy(src_hbm.at[pl.ds(off, CHUNK)], i_s, sem)
        pltpu.async_copy(dst_hbm.at[pl.ds(off, CHUNK)], i_d, sem)

    def drain_idx(i_s, i_d, sem):
        pltpu.make_async_copy(src_hbm.at[pl.ds(0, CHUNK)], i_s, sem).wait()
        pltpu.make_async_copy(dst_hbm.at[pl.ds(0, CHUNK)], i_d, sem).wait()

    def gather(i_s, i_d, rs, rd, sem):
        pltpu.async_copy(z_sh.at[i_s], rs, sem)
        pltpu.async_copy(z_sh.at[i_d], rd, sem)

    def drain_gather(i_s, i_d, rs, rd, sem):
        pltpu.make_async_copy(z_sh.at[i_s], rs, sem).wait()
        pltpu.make_async_copy(z_sh.at[i_d], rd, sem).wait()

    def drain_out(pr, sem):
        pltpu.make_async_copy(pr, x_hbm.at[pl.ds(0, CHUNK // 2)], sem).wait()

    issue_idx(0, isA, idA, isem_a)
    issue_idx(1, isB, idB, isem_b)
    drain_idx(isA, idA, isem_a)
    gather(isA, idA, rsA, rdA, gsem_a)

    def pair_body(k, carry):
        c0 = 2 * k
        last = N_PAIRS - 1

        drain_idx(isB, idB, isem_b)
        gather(isB, idB, rsB, rdB, gsem_b)

        drain_gather(isA, idA, rsA, rdA, gsem_a)

        @pl.when(k < last)
        def _():
            issue_idx(c0 + 2, isA, idA, isem_a)

        @pl.when(k > 0)
        def _():
            drain_out(prA, osem_a)

        _mul_chunk(rsA, rdA, prA)
        pltpu.async_copy(prA,
                         x_hbm.at[pl.ds((base + c0 * CHUNK) // 2, CHUNK // 2)],
                         osem_a)

        @pl.when(k < last)
        def _():
            drain_idx(isA, idA, isem_a)
            gather(isA, idA, rsA, rdA, gsem_a)

        drain_gather(isB, idB, rsB, rdB, gsem_b)

        @pl.when(k < last)
        def _():
            issue_idx(c0 + 3, isB, idB, isem_b)

        @pl.when(k > 0)
        def _():
            drain_out(prB, osem_b)

        _mul_chunk(rsB, rdB, prB)
        pltpu.async_copy(
            prB,
            x_hbm.at[pl.ds((base + (c0 + 1) * CHUNK) // 2, CHUNK // 2)],
            osem_b)
        return carry

    lax.fori_loop(0, N_PAIRS, pair_body, None, unroll=False)
    drain_out(prA, osem_a)
    drain_out(prB, osem_b)


def _gather_mul(zw, src, dst):
    mesh = plsc.VectorSubcoreMesh(core_axis_name="c", subcore_axis_name="s",
                                  num_cores=NC, num_subcores=NS)
    return pl.kernel(
        _sc_gather_mul,
        out_type=jax.ShapeDtypeStruct((E // 2, D), jnp.int32),
        mesh=mesh,
        compiler_params=pltpu.CompilerParams(use_tc_tiling_on_sc=False),
        scratch_types=[
            pltpu.VMEM((CHUNK,), jnp.int32),
            pltpu.VMEM((CHUNK,), jnp.int32),
            pltpu.VMEM((CHUNK,), jnp.int32),
            pltpu.VMEM((CHUNK,), jnp.int32),
            pltpu.VMEM((CHUNK, DW), jnp.int32),
            pltpu.VMEM((CHUNK, DW), jnp.int32),
            pltpu.VMEM((CHUNK // 2, D), jnp.int32),
            pltpu.VMEM((CHUNK, DW), jnp.int32),
            pltpu.VMEM((CHUNK, DW), jnp.int32),
            pltpu.VMEM((CHUNK // 2, D), jnp.int32),
            pltpu.VMEM_SHARED((N, DW), jnp.int32),
            pltpu.SemaphoreType.DMA,
            pltpu.SemaphoreType.DMA,
            pltpu.SemaphoreType.DMA,
            pltpu.SemaphoreType.DMA,
            pltpu.SemaphoreType.DMA,
            pltpu.SemaphoreType.DMA,
        ],
    )(src, dst, zw)


E2 = E // 2
E2_BLK = 1600


def _tc_mlp(xw_ref, w1a_ref, w1b_ref, b1_ref, w2_ref, b2_ref, o_ref):
    w = xw_ref[...]
    xe = _bc(w << 16, jnp.float32).astype(jnp.bfloat16)
    xo = _bc(w & _MHI, jnp.float32).astype(jnp.bfloat16)
    w1a = w1a_ref[...]
    w1b = w1b_ref[...]
    he = jnp.dot(xe[:, :DW], w1a, preferred_element_type=jnp.float32)
    he += jnp.dot(xo[:, :DW], w1b, preferred_element_type=jnp.float32)
    ho = jnp.dot(xe[:, DW:], w1a, preferred_element_type=jnp.float32)
    ho += jnp.dot(xo[:, DW:], w1b, preferred_element_type=jnp.float32)
    he = jnp.maximum(he + b1_ref[...], 0.0)
    ho = jnp.maximum(ho + b1_ref[...], 0.0)
    oe = jnp.sum(he * w2_ref[...], axis=1, keepdims=True) + b2_ref[...]
    oo = jnp.sum(ho * w2_ref[...], axis=1, keepdims=True) + b2_ref[...]
    o_ref[...] = jnp.concatenate([oe, oo], axis=1)


def _mlp(xw, W1a, W1b, b1, W2, b2):
    grid = (E2 // E2_BLK,)
    return pl.pallas_call(
        _tc_mlp,
        grid=grid,
        in_specs=[
            pl.BlockSpec((E2_BLK, D), lambda i: (i, 0)),
            pl.BlockSpec((DW, D), lambda i: (0, 0)),
            pl.BlockSpec((DW, D), lambda i: (0, 0)),
            pl.BlockSpec((1, D), lambda i: (0, 0)),
            pl.BlockSpec((1, D), lambda i: (0, 0)),
            pl.BlockSpec((1, 1), lambda i: (0, 0)),
        ],
        out_specs=pl.BlockSpec((E2_BLK, 2), lambda i: (i, 0)),
        out_shape=jax.ShapeDtypeStruct((E2, 2), jnp.float32),
    )(xw, W1a, W1b, b1.reshape(1, D), W2.reshape(1, D), b2.reshape(1, 1))


def kernel(z, edge, W1, b1, W2, b2):
    zb = z.astype(jnp.bfloat16)
    lo = _bc(zb[:, :DW], jnp.uint16).astype(jnp.int32)
    hi = _bc(zb[:, DW:], jnp.uint16).astype(jnp.int32)
    zw = lo | (hi << 16)
    xw = _gather_mul(zw, edge[0], edge[1])
    W1a = W1[:DW, :].astype(jnp.bfloat16)
    W1b = W1[DW:, :].astype(jnp.bfloat16)
    return _mlp(xw, W1a, W1b, b1, W2, b2).reshape(E, 1)
